# Initial kernel scaffold; baseline (speedup 1.0000x reference)
import jax
import jax.numpy as jnp
from jax.experimental import pallas as pl


def kernel(sequence, segment_label, token_table, segment_table, pe):
    raise NotImplementedError("write your pallas kernel here")


# speedup vs baseline: 1.1780x; 1.1780x over previous
import functools

import jax
import jax.numpy as jnp
from jax import lax
from jax.experimental import pallas as pl
from jax.experimental.pallas import tpu as pltpu
from jax.experimental.pallas import tpu_sc as plsc

B = 1024
S = 200
D = 64
N = B * S
NW = 32
PER_W = N // NW
C = 128
NCH = PER_W // C
L = 16


def _body(seq, lbl, tok, comb, out, idx_v, lbl_v, cidx_v, bufA, bufB, semA, semB):
    wid = lax.axis_index("s") * 2 + lax.axis_index("c")
    wbase = wid * PER_W

    def chunk(k, _):
        base = wbase + k * C
        pltpu.sync_copy(seq.at[pl.ds(base, C)], idx_v)
        pltpu.sync_copy(lbl.at[pl.ds(base, C)], lbl_v)

        for g in range(C // L):
            ids = idx_v[pl.ds(g * L, L)]
            lbs = lbl_v[pl.ds(g * L, L)]
            pos = lax.rem(base + g * L + lax.iota(jnp.int32, L), S)
            cidx = lbs * S + pos + jnp.where(ids == 0, 3 * S, 0)
            cidx_v[pl.ds(g * L, L)] = cidx

        ga = pltpu.async_copy(tok.at[idx_v], bufA, semA)
        gb = pltpu.async_copy(comb.at[cidx_v], bufB, semB)
        ga.wait()
        gb.wait()

        def add_row(r, _):
            for j in range(D // L):
                bufA[r, pl.ds(j * L, L)] = (
                    bufA[r, pl.ds(j * L, L)] + bufB[r, pl.ds(j * L, L)]
                )
            return 0

        lax.fori_loop(0, C, add_row, 0)
        pltpu.sync_copy(bufA, out.at[pl.ds(base, C)])
        return 0

    lax.fori_loop(0, NCH, chunk, 0)


def kernel(sequence, segment_label, token_table, segment_table, pe):
    seq = sequence.reshape(N).astype(jnp.int32)
    lbl = segment_label.reshape(N).astype(jnp.int32)

    seg0 = segment_table.at[0].set(0.0)
    base_tab = (seg0[:, None, :] + pe[0][None, :, :]).reshape(3 * S, D)
    comb = jnp.concatenate([base_tab, base_tab - token_table[0][None, :]], axis=0)

    run = pl.kernel(
        _body,
        out_type=jax.ShapeDtypeStruct((N, D), jnp.float32),
        mesh=plsc.VectorSubcoreMesh(core_axis_name="c", subcore_axis_name="s"),
        compiler_params=pltpu.CompilerParams(use_tc_tiling_on_sc=False),
        scratch_types=[
            pltpu.VMEM((C,), jnp.int32),
            pltpu.VMEM((C,), jnp.int32),
            pltpu.VMEM((C,), jnp.int32),
            pltpu.VMEM((C, D), jnp.float32),
            pltpu.VMEM((C, D), jnp.float32),
            pltpu.SemaphoreType.DMA,
            pltpu.SemaphoreType.DMA,
        ],
    )
    out = run(seq, lbl, token_table, comb)
    return out.reshape(B, S, D)


# speedup vs baseline: 1.1800x; 1.0017x over previous
import jax
import jax.numpy as jnp
from jax import lax
from jax.experimental import pallas as pl
from jax.experimental.pallas import tpu as pltpu
from jax.experimental.pallas import tpu_sc as plsc

B = 1024
S = 200
D = 64
N = B * S
NW = 32
PER_W = N // NW
G = 2
C = G * 128
NCH = PER_W // C
NBUF = 2
RPW = PER_W // 128
L = 16


def _body(seq, lbl, tok, comb, out,
          idx_v, lbl_v, cidx_v, bufA, bufB, in_s, ga_s, gb_s, out_s):
    wid = lax.axis_index("s") * 2 + lax.axis_index("c")
    rbase = wid * RPW

    def fire_in(k, b):
        r = rbase + k * G
        pltpu.async_copy(seq.at[pl.ds(r, G)], idx_v.at[b], in_s.at[b])
        pltpu.async_copy(lbl.at[pl.ds(r, G)], lbl_v.at[b], in_s.at[b])

    def wait_in(b):
        pltpu.make_async_copy(seq.at[pl.ds(0, G)], idx_v.at[b], in_s.at[b]).wait()
        pltpu.make_async_copy(lbl.at[pl.ds(0, G)], lbl_v.at[b], in_s.at[b]).wait()

    def compute_cidx(k, b):
        for i in range(G):
            row = (rbase + k * G + i) * 128
            for g in range(128 // L):
                ids = idx_v[b, i, pl.ds(g * L, L)]
                lbs = lbl_v[b, i, pl.ds(g * L, L)]
                pos = lax.rem(row + g * L + lax.iota(jnp.int32, L), S)
                cidx_v[b, i, pl.ds(g * L, L)] = (
                    lbs * S + pos + jnp.where(ids == 0, 3 * S, 0)
                )

    def fire_gathers(b):
        for i in range(G):
            pltpu.async_copy(tok.at[idx_v.at[b, i]], bufA.at[b, i], ga_s.at[b])
            pltpu.async_copy(comb.at[cidx_v.at[b, i]], bufB.at[b, i], gb_s.at[b])

    def wait_gathers(b):
        for i in range(G):
            pltpu.make_async_copy(tok.at[idx_v.at[b, i]], bufA.at[b, i], ga_s.at[b]).wait()
            pltpu.make_async_copy(comb.at[cidx_v.at[b, i]], bufB.at[b, i], gb_s.at[b]).wait()

    def add_chunk(b):
        @pl.loop(0, 128)
        def _(r):
            for i in range(G):
                for j in range(D // L):
                    bufA[b, i, r, pl.ds(j * L, L)] = (
                        bufA[b, i, r, pl.ds(j * L, L)]
                        + bufB[b, i, r, pl.ds(j * L, L)]
                    )

    def fire_out(k, b):
        pltpu.async_copy(bufA.at[b], out.at[pl.ds(rbase + k * G, G)], out_s.at[b])

    def wait_out(b):
        pltpu.make_async_copy(bufA.at[b], out.at[pl.ds(0, G)], out_s.at[b]).wait()

    fire_in(0, 0)
    wait_in(0)
    compute_cidx(0, 0)
    fire_gathers(0)
    fire_in(1, 1)

    def step(k, _):
        b = lax.rem(k, 2)
        p = 1 - b
        wait_in(b)
        compute_cidx(k, b)

        @pl.when(k >= 2)
        def _():
            wait_out(b)

        fire_gathers(b)
        wait_gathers(p)

        @pl.when(k + 1 < NCH)
        def _():
            fire_in(k + 1, p)

        add_chunk(p)
        fire_out(k - 1, p)
        return 0

    lax.fori_loop(1, NCH, step, 0)

    bl = (NCH - 1) % 2
    wait_gathers(bl)
    add_chunk(bl)
    fire_out(NCH - 1, bl)
    wait_out(1 - bl)
    wait_out(bl)


def kernel(sequence, segment_label, token_table, segment_table, pe):
    seq = sequence.reshape(N // 128, 128).astype(jnp.int32)
    lbl = segment_label.reshape(N // 128, 128).astype(jnp.int32)

    seg0 = segment_table.at[0].set(0.0)
    base_tab = (seg0[:, None, :] + pe[0][None, :, :]).reshape(3 * S, D)
    comb = jnp.concatenate([base_tab, base_tab - token_table[0][None, :]], axis=0)

    run = pl.kernel(
        _body,
        out_type=jax.ShapeDtypeStruct((N // 128, 128, D), jnp.float32),
        mesh=plsc.VectorSubcoreMesh(core_axis_name="c", subcore_axis_name="s"),
        compiler_params=pltpu.CompilerParams(use_tc_tiling_on_sc=False),
        scratch_types=[
            pltpu.VMEM((NBUF, G, 128), jnp.int32),
            pltpu.VMEM((NBUF, G, 128), jnp.int32),
            pltpu.VMEM((NBUF, G, 128), jnp.int32),
            pltpu.VMEM((NBUF, G, 128, D), jnp.float32),
            pltpu.VMEM((NBUF, G, 128, D), jnp.float32),
            pltpu.SemaphoreType.DMA((NBUF,)),
            pltpu.SemaphoreType.DMA((NBUF,)),
            pltpu.SemaphoreType.DMA((NBUF,)),
            pltpu.SemaphoreType.DMA((NBUF,)),
        ],
    )
    out = run(seq, lbl, token_table, comb)
    return out.reshape(B, S, D)
